# Initial kernel scaffold; baseline (speedup 1.0000x reference)
import jax
import jax.numpy as jnp
from jax.experimental import pallas as pl


def kernel(users, items, edge_index, edge_weight, user_table, item_table):
    raise NotImplementedError("write your pallas kernel here")


# speedup vs baseline: 3.3869x; 3.3869x over previous
import functools

import jax
import jax.numpy as jnp
from jax import lax
from jax.experimental import pallas as pl
from jax.experimental.pallas import tpu as pltpu
from jax.experimental.pallas import tpu_sc as plsc

N_USERS = 30000
N_ITEMS = 20000
N_NODES = N_USERS + N_ITEMS
D = 64
E = 800000
EH = E // 2
LAYERS = 3

NC = 2
NS = 16
LANES = 16

PAD_ROWS = 80
ITEM_BASE = N_USERS + PAD_ROWS
N_PAD = N_USERS + PAD_ROWS + N_ITEMS

CHUNK = 128
CHUNKS_PER_TILE = -(-EH // (NS * CHUNK))
EDGES_PER_TILE = CHUNK * CHUNKS_PER_TILE
EH_PAD = EDGES_PER_TILE * NS
ACC_ROWS = ITEM_BASE
ZROWS = ACC_ROWS // NS

U_RPT, U_LAST = 1872, N_USERS - 15 * 1872
I_RPT, I_LAST = 1248, N_ITEMS - 15 * 1248

_mesh = plsc.VectorSubcoreMesh(core_axis_name="c", subcore_axis_name="s")


@functools.partial(
    pl.kernel,
    mesh=_mesh,
    compiler_params=pltpu.CompilerParams(needs_layout_passes=False, use_tc_tiling_on_sc=False),
    out_type=jax.ShapeDtypeStruct((N_PAD, D), jnp.float32),
    scratch_types=[
        pltpu.VMEM_SHARED((ACC_ROWS, D), jnp.float32),
        pltpu.VMEM((CHUNK,), jnp.int32),
        pltpu.VMEM((CHUNK,), jnp.int32),
        pltpu.VMEM((CHUNK,), jnp.float32),
        pltpu.VMEM((CHUNK, D), jnp.float32),
        pltpu.SemaphoreType.DMA,
    ],
)
def _propagate(emb, srcp, dstp, wp, zrows, out, acc, src_v, dst_v, w_v, rows_v, sem):
    c = lax.axis_index("c")
    s = lax.axis_index("s")
    base_node = (1 - c) * N_USERS
    edge_base = c * EH_PAD + s * EDGES_PER_TILE

    pltpu.sync_copy(zrows, acc.at[pl.ds(s * ZROWS, ZROWS)])
    plsc.subcore_barrier()

    def chunk_body(g, carry):
        base = edge_base + g * CHUNK
        pltpu.sync_copy(srcp.at[pl.ds(base, CHUNK)], src_v)
        pltpu.sync_copy(dstp.at[pl.ds(base, CHUNK)], dst_v)
        pltpu.sync_copy(wp.at[pl.ds(base, CHUNK)], w_v)
        pltpu.async_copy(emb.at[src_v], rows_v, sem).wait()

        lane = lax.iota(jnp.int32, LANES)

        def grp_body(gg, carry2):
            o = gg * LANES
            dst_v[pl.ds(o, LANES)] = dst_v[pl.ds(o, LANES)] - base_node
            wv = w_v[pl.ds(o, LANES)]
            for e16 in range(LANES):
                e = o + e16
                we = jnp.sum(jnp.where(lane == e16, wv, 0.0))
                for j in range(D // LANES):
                    rows_v[e, pl.ds(j * LANES, LANES)] = (
                        rows_v[e, pl.ds(j * LANES, LANES)] * we
                    )
            return carry2

        lax.fori_loop(0, CHUNK // LANES, grp_body, 0)
        pltpu.sync_copy(rows_v, acc.at[dst_v], add=True)
        return carry

    lax.fori_loop(0, CHUNKS_PER_TILE, chunk_body, 0)
    plsc.subcore_barrier()

    @pl.when((c == 0) & (s < 15))
    def _():
        pltpu.sync_copy(
            acc.at[pl.ds(s * I_RPT, I_RPT)], out.at[pl.ds(ITEM_BASE + s * I_RPT, I_RPT)]
        )

    @pl.when((c == 0) & (s == 15))
    def _():
        pltpu.sync_copy(
            acc.at[pl.ds(15 * I_RPT, I_LAST)],
            out.at[pl.ds(ITEM_BASE + 15 * I_RPT, I_LAST)],
        )

    @pl.when((c == 1) & (s < 15))
    def _():
        pltpu.sync_copy(acc.at[pl.ds(s * U_RPT, U_RPT)], out.at[pl.ds(s * U_RPT, U_RPT)])

    @pl.when((c == 1) & (s == 15))
    def _():
        pltpu.sync_copy(
            acc.at[pl.ds(15 * U_RPT, U_LAST)], out.at[pl.ds(15 * U_RPT, U_LAST)]
        )


def _make_forward(batch):
    bt = batch // (NC * NS)

    @functools.partial(
        pl.kernel,
        mesh=_mesh,
        compiler_params=pltpu.CompilerParams(needs_layout_passes=False, use_tc_tiling_on_sc=False),
        out_type=jax.ShapeDtypeStruct((batch,), jnp.float32),
        scratch_types=[
            pltpu.VMEM((bt,), jnp.int32),
            pltpu.VMEM((bt,), jnp.int32),
            pltpu.VMEM((bt, D), jnp.float32),
            pltpu.VMEM((bt, D), jnp.float32),
            pltpu.VMEM((bt, D), jnp.float32),
            pltpu.VMEM((bt,), jnp.float32),
            pltpu.SemaphoreType.DMA,
        ],
    )
    def _forward(e0, e1, e2, e3, users, items, gamma, uidx, iidx, rows, uacc, iacc, gam, sem):
        c = lax.axis_index("c")
        s = lax.axis_index("s")
        b0 = (s * NC + c) * bt
        pltpu.sync_copy(users.at[pl.ds(b0, bt)], uidx)
        pltpu.sync_copy(items.at[pl.ds(b0, bt)], iidx)

        def off_body(gg, carry):
            o = gg * LANES
            iidx[pl.ds(o, LANES)] = iidx[pl.ds(o, LANES)] + ITEM_BASE
            return carry

        lax.fori_loop(0, bt // LANES, off_body, 0)

        for idx, dacc in ((uidx, uacc), (iidx, iacc)):
            for t, tab in enumerate((e0, e1, e2, e3)):
                pltpu.async_copy(tab.at[idx], rows, sem).wait()

                def acc_body(e, carry, t=t, dacc=dacc):
                    for j in range(D // LANES):
                        sl = pl.ds(j * LANES, LANES)
                        v = rows[e, sl]
                        if t:
                            v = dacc[e, sl] + v
                        dacc[e, sl] = v
                    return carry

                lax.fori_loop(0, bt, acc_body, 0)

        scale = 1.0 / float((LAYERS + 1) ** 2)
        lane = lax.iota(jnp.int32, LANES)

        def dot_body(g, carry):
            o = g * LANES
            accv = jnp.zeros((LANES,), jnp.float32)
            for e16 in range(LANES):
                e = o + e16
                ps = jnp.zeros((LANES,), jnp.float32)
                for j in range(D // LANES):
                    sl = pl.ds(j * LANES, LANES)
                    ps = ps + uacc[e, sl] * iacc[e, sl]
                tot = jnp.sum(ps) * scale
                accv = jnp.where(lane == e16, tot, accv)
            gam[pl.ds(o, LANES)] = accv
            return carry

        lax.fori_loop(0, bt // LANES, dot_body, 0)
        pltpu.sync_copy(gam, gamma.at[pl.ds(b0, bt)])

    return _forward


def kernel(users, items, edge_index, edge_weight, user_table, item_table):
    emb0 = jnp.concatenate(
        [user_table, jnp.zeros((PAD_ROWS, D), jnp.float32), item_table], axis=0
    )
    src = edge_index[0]
    dst = edge_index[1]
    src = jnp.where(src >= N_USERS, src + PAD_ROWS, src)
    pad = EH_PAD - EH
    zi = jnp.zeros((pad,), jnp.int32)
    zf = jnp.zeros((pad,), jnp.float32)
    srcp = jnp.concatenate([src[:EH], zi, src[EH:], zi])
    dstp = jnp.concatenate(
        [dst[:EH], jnp.full((pad,), N_USERS, jnp.int32), dst[EH:], zi]
    )
    wp = jnp.concatenate([edge_weight[:EH], zf, edge_weight[EH:], zf])
    zrows = jnp.zeros((ZROWS, D), jnp.float32)

    e0 = emb0
    e1 = _propagate(e0, srcp, dstp, wp, zrows)
    e2 = _propagate(e1, srcp, dstp, wp, zrows)
    e3 = _propagate(e2, srcp, dstp, wp, zrows)
    fwd = _make_forward(users.shape[0])
    return fwd(e0, e1, e2, e3, users, items)
